# baseline (speedup 1.0000x reference)
import functools

import jax
import jax.numpy as jnp
from jax import lax
from jax.experimental import pallas as pl
from jax.experimental.pallas import tpu as pltpu
from jax.experimental.pallas import tpu_sc as plsc

N = 50000
E = 800000
D_IN = 1433
D_HID = 112
D_OUT = 7

NC = 2
NS = 16
B = 128
NP = 50048
RPT = NP // NS
E_PAD = 835584
NB_FULL = E_PAD // (NS * B)
NB_HALF = E_PAD // (NC * NS * B)
CW = 32
NCHUNK = 4
W3P = 16
RB = 400

@functools.lru_cache(maxsize=1)
def _mesh():
  return plsc.VectorSubcoreMesh(
      core_axis_name="c", subcore_axis_name="s", num_cores=NC, num_subcores=NS)


def _deg_kernel(pk1_hbm, ones_hbm, zeros_hbm, out_hbm, idx_v, ones_v, acc):
  core = lax.axis_index("c")
  sub = lax.axis_index("s")
  tile = core * NS + sub
  pltpu.sync_copy(zeros_hbm.at[pl.ds(sub * RPT, RPT)],
                  acc.at[pl.ds(sub * RPT, RPT)])
  pltpu.sync_copy(ones_hbm, ones_v)
  plsc.subcore_barrier()

  def body(t, carry):
    row = tile * NB_HALF + t
    pltpu.sync_copy(pk1_hbm.at[row], idx_v)
    pltpu.sync_copy(ones_v, acc.at[idx_v.at[1]], add=True)
    return carry

  lax.fori_loop(0, NB_HALF, body, 0)
  plsc.subcore_barrier()
  pltpu.sync_copy(acc.at[pl.ds(sub * RPT, RPT)],
                  out_hbm.at[pl.ds(core * NP + sub * RPT, RPT)])


def _sc_deg(pk1, ones16, zeros16):
  return pl.kernel(
      _deg_kernel,
      out_type=jax.ShapeDtypeStruct((NC * NP, W3P), jnp.float32),
      mesh=_mesh(),
      compiler_params=pltpu.CompilerParams(use_tc_tiling_on_sc=False),
      scratch_types=[
          pltpu.VMEM((2, B), jnp.int32),
          pltpu.VMEM((B, W3P), jnp.float32),
          pltpu.VMEM_SHARED((NP, W3P), jnp.float32),
      ],
  )(pk1, ones16, zeros16)


KI = 6
GA = 3


def _edge_pipeline(pk_hbm, ytab_hbm, acc, ibuf, rows_v, isem, gsems, ssems,
                   rowbase, nb):
  ng = nb // KI

  def idx_start(g, buf):
    pltpu.async_copy(pk_hbm.at[pl.ds(rowbase + g * KI, KI)], ibuf.at[buf],
                     isem)

  def idx_wait(buf):
    pltpu.make_async_copy(pk_hbm.at[pl.ds(rowbase, KI)], ibuf.at[buf],
                          isem).wait()

  def gather_start(buf, k, slot):
    pltpu.async_copy(ytab_hbm.at[ibuf.at[buf, k, 0]], rows_v.at[slot],
                     gsems[slot % 3])

  def gather_wait(buf, k, slot):
    pltpu.make_async_copy(ytab_hbm.at[ibuf.at[buf, k, 0]], rows_v.at[slot],
                          gsems[slot % 3]).wait()

  def scatter_start(buf, k, slot):
    pltpu.async_copy(rows_v.at[slot], acc.at[ibuf.at[buf, k, 1]],
                     ssems[slot], add=True)

  def scatter_wait(buf, k, slot):
    pltpu.make_async_copy(rows_v.at[slot], acc.at[ibuf.at[buf, k, 1]],
                          ssems[slot]).wait()

  idx_start(0, 0)
  idx_wait(0)
  for p in range(GA):
    gather_start(0, p, p)

  def outer(g, carry):
    gm = lax.rem(g, 3)
    gn = lax.rem(g + 1, 3)

    @pl.when(g + 1 < ng)
    def _():
      idx_start(g + 1, gn)

    for k in range(KI):
      t = g * KI + k

      @pl.when(t + GA < nb)
      def _():
        @pl.when(t + GA - KI >= 0)
        def _():
          scatter_wait(gm, (k + GA) % KI, (k + GA) % KI)
        if k < KI - GA:
          gather_start(gm, k + GA, (k + GA) % KI)
        else:
          if k == KI - GA:
            idx_wait(gn)
          gather_start(gn, k + GA - KI, (k + GA) % KI)

      gather_wait(gm, k, k)
      scatter_start(gm, k, k)
    return carry

  lax.fori_loop(0, ng, outer, 0)

  for k in range(KI):
    scatter_wait(0, k, k)


def _agg32_kernel(ytab_hbm, pk4_hbm, zeros_hbm, out_hbm, ibuf, rows_v, acc,
                  isem, g0, g1, g2, s0, s1, s2, s3, s4, s5):
  core = lax.axis_index("c")
  sub = lax.axis_index("s")
  for j in range(NCHUNK // NC):
    c = core * (NCHUNK // NC) + j
    pltpu.sync_copy(zeros_hbm.at[pl.ds(sub * RPT, RPT)],
                    acc.at[pl.ds(sub * RPT, RPT)])
    plsc.subcore_barrier()
    rowbase = (c * NS + sub) * NB_FULL
    _edge_pipeline(pk4_hbm, ytab_hbm, acc, ibuf, rows_v, isem,
                   (g0, g1, g2), (s0, s1, s2, s3, s4, s5),
                   rowbase, NB_FULL)
    plsc.subcore_barrier()
    pltpu.sync_copy(acc.at[pl.ds(sub * RPT, RPT)],
                    out_hbm.at[pl.ds(c * NP + sub * RPT, RPT)])
    plsc.subcore_barrier()


def _sc_agg32(ytab, pk4, zeros32):
  return pl.kernel(
      _agg32_kernel,
      out_type=jax.ShapeDtypeStruct((NCHUNK * NP, CW), jnp.float32),
      mesh=_mesh(),
      compiler_params=pltpu.CompilerParams(use_tc_tiling_on_sc=False),
      scratch_types=[
          pltpu.VMEM((3, KI, 2, B), jnp.int32),
          pltpu.VMEM((KI, B, CW), jnp.float32),
          pltpu.VMEM_SHARED((NP, CW), jnp.float32),
          pltpu.SemaphoreType.DMA,
          pltpu.SemaphoreType.DMA,
          pltpu.SemaphoreType.DMA,
          pltpu.SemaphoreType.DMA,
          pltpu.SemaphoreType.DMA,
          pltpu.SemaphoreType.DMA,
          pltpu.SemaphoreType.DMA,
          pltpu.SemaphoreType.DMA,
          pltpu.SemaphoreType.DMA,
          pltpu.SemaphoreType.DMA,
      ],
  )(ytab, pk4, zeros32)


def _agg16_kernel(ytab_hbm, pk1_hbm, zeros_hbm, out_hbm, ibuf, rows_v, acc,
                  isem, g0, g1, g2, s0, s1, s2, s3, s4, s5):
  core = lax.axis_index("c")
  sub = lax.axis_index("s")
  tile = core * NS + sub
  pltpu.sync_copy(zeros_hbm.at[pl.ds(sub * RPT, RPT)],
                  acc.at[pl.ds(sub * RPT, RPT)])
  plsc.subcore_barrier()
  _edge_pipeline(pk1_hbm, ytab_hbm, acc, ibuf, rows_v, isem,
                 (g0, g1, g2), (s0, s1, s2, s3, s4, s5),
                 tile * NB_HALF, NB_HALF)
  plsc.subcore_barrier()
  pltpu.sync_copy(acc.at[pl.ds(sub * RPT, RPT)],
                  out_hbm.at[pl.ds(core * NP + sub * RPT, RPT)])


def _sc_agg16(ytab, pk1, zeros16):
  return pl.kernel(
      _agg16_kernel,
      out_type=jax.ShapeDtypeStruct((NC * NP, W3P), jnp.float32),
      mesh=_mesh(),
      compiler_params=pltpu.CompilerParams(use_tc_tiling_on_sc=False),
      scratch_types=[
          pltpu.VMEM((3, KI, 2, B), jnp.int32),
          pltpu.VMEM((KI, B, W3P), jnp.float32),
          pltpu.VMEM_SHARED((NP, W3P), jnp.float32),
          pltpu.SemaphoreType.DMA,
          pltpu.SemaphoreType.DMA,
          pltpu.SemaphoreType.DMA,
          pltpu.SemaphoreType.DMA,
          pltpu.SemaphoreType.DMA,
          pltpu.SemaphoreType.DMA,
          pltpu.SemaphoreType.DMA,
          pltpu.SemaphoreType.DMA,
          pltpu.SemaphoreType.DMA,
          pltpu.SemaphoreType.DMA,
      ],
  )(ytab, pk1, zeros16)


def _chunked(y):
  zpad = jnp.zeros((y.shape[0], CW - D_HID // NCHUNK), jnp.float32)
  parts = []
  for c in range(NCHUNK):
    parts.append(
        jnp.concatenate([y[:, c * 28:(c + 1) * 28], zpad], axis=1)[None])
  return jnp.concatenate(parts, axis=0)


def _norm_from_deg(degp):
  deg = degp[0, :, 0] + degp[1, :, 0]
  return jnp.where(deg > 0.0, lax.rsqrt(jnp.maximum(deg, 1.0)), 0.0)


def _tc1_body(degp_ref, x_ref, w_ref, y0_ref, norm_ref):
  norm = _norm_from_deg(degp_ref[...])
  xs = x_ref[...] * norm[:, None]
  y = jnp.dot(xs, w_ref[...], preferred_element_type=jnp.float32,
              precision=lax.Precision.HIGHEST)
  y0_ref[...] = _chunked(y)
  norm_ref[...] = jnp.broadcast_to(norm[:, None], (RB, 8))


def _tc1(degp, features, w0):
  grid = N // RB
  return pl.pallas_call(
      _tc1_body,
      grid=(grid,),
      in_specs=[
          pl.BlockSpec((2, RB, W3P), lambda i: (0, i, 0)),
          pl.BlockSpec((RB, D_IN), lambda i: (i, 0)),
          pl.BlockSpec((D_IN, D_HID), lambda i: (0, 0)),
      ],
      out_specs=[
          pl.BlockSpec((NCHUNK, RB, CW), lambda i: (0, i, 0)),
          pl.BlockSpec((RB, 8), lambda i: (i, 0)),
      ],
      out_shape=[
          jax.ShapeDtypeStruct((NCHUNK, N, CW), jnp.float32),
          jax.ShapeDtypeStruct((N, 8), jnp.float32),
      ],
  )(degp, features, w0)


def _mid_body(residual, use_act, agg_ref, norm_ref, w_ref, *rest):
  if residual:
    xin_ref = rest[0]
    rest = rest[1:]
  x_ref, y_ref = rest
  a = agg_ref[...]
  a112 = jnp.concatenate([a[c, :, :28] for c in range(NCHUNK)], axis=1)
  n = norm_ref[:, 0]
  h = a112 * n[:, None]
  if use_act:
    h = jnp.tanh(h)
  if residual:
    h = h + xin_ref[...]
  y = jnp.dot(h * n[:, None], w_ref[...], preferred_element_type=jnp.float32,
              precision=lax.Precision.HIGHEST)
  x_ref[...] = h
  y_ref[...] = _chunked(y)


def _tc_mid(agg, normw, w, xin):
  grid = N // RB
  residual = xin is not None
  body = functools.partial(_mid_body, residual, not residual)
  in_specs = [
      pl.BlockSpec((NCHUNK, RB, CW), lambda i: (0, i, 0)),
      pl.BlockSpec((RB, 8), lambda i: (i, 0)),
      pl.BlockSpec((D_HID, D_HID), lambda i: (0, 0)),
  ]
  args = [agg, normw, w]
  if residual:
    in_specs.append(pl.BlockSpec((RB, D_HID), lambda i: (i, 0)))
    args.append(xin)
  return pl.pallas_call(
      body,
      grid=(grid,),
      in_specs=in_specs,
      out_specs=[
          pl.BlockSpec((RB, D_HID), lambda i: (i, 0)),
          pl.BlockSpec((NCHUNK, RB, CW), lambda i: (0, i, 0)),
      ],
      out_shape=[
          jax.ShapeDtypeStruct((N, D_HID), jnp.float32),
          jax.ShapeDtypeStruct((NCHUNK, N, CW), jnp.float32),
      ],
  )(*args)


def _tc4_body(agg_ref, norm_ref, xin_ref, w_ref, y_ref):
  a = agg_ref[...]
  a112 = jnp.concatenate([a[c, :, :28] for c in range(NCHUNK)], axis=1)
  n = norm_ref[:, 0]
  h = a112 * n[:, None] + xin_ref[...]
  y_ref[...] = jnp.dot(h * n[:, None], w_ref[...],
                       preferred_element_type=jnp.float32,
                       precision=lax.Precision.HIGHEST)


def _tc4(agg, normw, xin, w3p):
  grid = N // RB
  return pl.pallas_call(
      _tc4_body,
      grid=(grid,),
      in_specs=[
          pl.BlockSpec((NCHUNK, RB, CW), lambda i: (0, i, 0)),
          pl.BlockSpec((RB, 8), lambda i: (i, 0)),
          pl.BlockSpec((RB, D_HID), lambda i: (i, 0)),
          pl.BlockSpec((D_HID, W3P), lambda i: (0, 0)),
      ],
      out_specs=pl.BlockSpec((RB, W3P), lambda i: (i, 0)),
      out_shape=jax.ShapeDtypeStruct((N, W3P), jnp.float32),
  )(agg, normw, xin, w3p)


def _tc5_body(aggp_ref, norm_ref, out_ref):
  a = aggp_ref[0] + aggp_ref[1]
  out_ref[...] = a * norm_ref[:, :1]


def _tc5(aggp, normw):
  grid = N // RB
  return pl.pallas_call(
      _tc5_body,
      grid=(grid,),
      in_specs=[
          pl.BlockSpec((2, RB, W3P), lambda i: (0, i, 0)),
          pl.BlockSpec((RB, 8), lambda i: (i, 0)),
      ],
      out_specs=pl.BlockSpec((RB, W3P), lambda i: (i, 0)),
      out_shape=jax.ShapeDtypeStruct((N, W3P), jnp.float32),
  )(aggp, normw)


@jax.jit
def kernel(features, edge_index, W0, W1, W2, W3):
  src = edge_index[0].astype(jnp.int32)
  dst = edge_index[1].astype(jnp.int32)
  pad_e = E_PAD - E
  srcp = jnp.concatenate([src, jnp.zeros((pad_e,), jnp.int32)])
  dstp = jnp.concatenate([dst, jnp.full((pad_e,), N, jnp.int32)])

  src4 = srcp[None, :] + (jnp.arange(NCHUNK, dtype=jnp.int32) * N)[:, None]
  s4 = src4.reshape(NCHUNK, NS, NB_FULL, B)
  d4 = jnp.broadcast_to(dstp.reshape(1, NS, NB_FULL, B), s4.shape)
  pk4 = jnp.stack([s4, d4], axis=3).reshape(NCHUNK * NS * NB_FULL, 2, B)

  s1 = srcp.reshape(NC * NS, NB_HALF, B)
  d1 = dstp.reshape(NC * NS, NB_HALF, B)
  pk1 = jnp.stack([s1, d1], axis=2).reshape(NC * NS * NB_HALF, 2, B)

  zeros32 = jnp.zeros((NP, CW), jnp.float32)
  zeros16 = jnp.zeros((NP, W3P), jnp.float32)
  ones16 = jnp.ones((B, W3P), jnp.float32)
  w3p = jnp.pad(W3, ((0, 0), (0, W3P - D_OUT)))

  degp = _sc_deg(pk1, ones16, zeros16).reshape(NC, NP, W3P)
  y0, normw = _tc1(degp, features, W0)
  agg0 = _sc_agg32(y0.reshape(NCHUNK * N, CW), pk4, zeros32)
  x1, y1 = _tc_mid(agg0.reshape(NCHUNK, NP, CW), normw, W1, None)
  agg1 = _sc_agg32(y1.reshape(NCHUNK * N, CW), pk4, zeros32)
  x2, y2 = _tc_mid(agg1.reshape(NCHUNK, NP, CW), normw, W2, x1)
  agg2 = _sc_agg32(y2.reshape(NCHUNK * N, CW), pk4, zeros32)
  y3 = _tc4(agg2.reshape(NCHUNK, NP, CW), normw, x2, w3p)
  aggp3 = _sc_agg16(y3, pk1, zeros16)
  out = _tc5(aggp3.reshape(NC, NP, W3P), normw)
  return out[:, :D_OUT]


# speedup vs baseline: 1.2900x; 1.2900x over previous
import functools

import jax
import jax.numpy as jnp
from jax import lax
from jax.experimental import pallas as pl
from jax.experimental.pallas import tpu as pltpu
from jax.experimental.pallas import tpu_sc as plsc

N = 50000
E = 800000
D_IN = 1433
D_HID = 112
D_OUT = 7

NC = 2
NS = 16
B = 128
NP = 50048
RPT = NP // NS
E_PAD = 819200
NB_FULL = E_PAD // (NS * B)
NB_HALF = E_PAD // (NC * NS * B)
CW = 32
NCHUNK = 4
W3P = 16
RB = 400

@functools.lru_cache(maxsize=1)
def _mesh():
  return plsc.VectorSubcoreMesh(
      core_axis_name="c", subcore_axis_name="s", num_cores=NC, num_subcores=NS)


def _deg_kernel(pk1_hbm, ones_hbm, zeros_hbm, out_hbm, idx_v, ones_v, acc):
  core = lax.axis_index("c")
  sub = lax.axis_index("s")
  tile = core * NS + sub
  pltpu.sync_copy(zeros_hbm.at[pl.ds(sub * RPT, RPT)],
                  acc.at[pl.ds(sub * RPT, RPT)])
  pltpu.sync_copy(ones_hbm, ones_v)
  plsc.subcore_barrier()

  def body(t, carry):
    row = tile * NB_HALF + t
    pltpu.sync_copy(pk1_hbm.at[row], idx_v)
    pltpu.sync_copy(ones_v, acc.at[idx_v.at[1]], add=True)
    return carry

  lax.fori_loop(0, NB_HALF, body, 0)
  plsc.subcore_barrier()
  pltpu.sync_copy(acc.at[pl.ds(sub * RPT, RPT)],
                  out_hbm.at[pl.ds(core * NP + sub * RPT, RPT)])


def _sc_deg(pk1, ones16, zeros16):
  return pl.kernel(
      _deg_kernel,
      out_type=jax.ShapeDtypeStruct((NC * NP, W3P), jnp.float32),
      mesh=_mesh(),
      compiler_params=pltpu.CompilerParams(use_tc_tiling_on_sc=False),
      scratch_types=[
          pltpu.VMEM((2, B), jnp.int32),
          pltpu.VMEM((B, W3P), jnp.float32),
          pltpu.VMEM_SHARED((NP, W3P), jnp.float32),
      ],
  )(pk1, ones16, zeros16)


KI = 8
GD = 4


def _edge_pipeline(pk_hbm, ytab_hbm, acc, ibuf, rows_v, isem, gsems, rowbase,
                   nb):
  ng = nb // KI

  def idx_start(g, buf):
    pltpu.async_copy(pk_hbm.at[pl.ds(rowbase + g * KI, KI)], ibuf.at[buf],
                     isem)

  def idx_wait(buf):
    pltpu.make_async_copy(pk_hbm.at[pl.ds(rowbase, KI)], ibuf.at[buf],
                          isem).wait()

  def gather_start(buf, k, slot):
    pltpu.async_copy(ytab_hbm.at[ibuf.at[buf, k, 0]], rows_v.at[slot],
                     gsems[slot])

  def gather_wait(buf, k, slot):
    pltpu.make_async_copy(ytab_hbm.at[ibuf.at[buf, k, 0]], rows_v.at[slot],
                          gsems[slot]).wait()

  idx_start(0, 0)
  idx_wait(0)
  gather_start(0, 0, 0)
  gather_start(0, 1, 1)
  gather_start(0, 2, 2)

  def outer(g, carry):
    gm = lax.rem(g, 2)
    gn = lax.rem(g + 1, 2)

    @pl.when(g + 1 < ng)
    def _():
      idx_start(g + 1, gn)

    for k in range(KI):
      t = g * KI + k

      @pl.when(t + 3 < nb)
      def _():
        if k < KI - 3:
          gather_start(gm, k + 3, (k + 3) % GD)
        else:
          if k == KI - 3:
            idx_wait(gn)
          gather_start(gn, k + 3 - KI, (k + 3) % GD)

      gather_wait(gm, k, k % GD)
      pltpu.sync_copy(rows_v.at[k % GD], acc.at[ibuf.at[gm, k, 1]], add=True)
    return carry

  lax.fori_loop(0, ng, outer, 0)


def _agg32_kernel(ytab_hbm, pk4_hbm, zeros_hbm, out_hbm, ibuf, rows_v, acc,
                  isem, g0, g1, g2, g3):
  core = lax.axis_index("c")
  sub = lax.axis_index("s")
  for j in range(NCHUNK // NC):
    c = core * (NCHUNK // NC) + j
    pltpu.sync_copy(zeros_hbm.at[pl.ds(sub * RPT, RPT)],
                    acc.at[pl.ds(sub * RPT, RPT)])
    plsc.subcore_barrier()
    rowbase = (c * NS + sub) * NB_FULL
    _edge_pipeline(pk4_hbm, ytab_hbm, acc, ibuf, rows_v, isem,
                   (g0, g1, g2, g3), rowbase, NB_FULL)
    plsc.subcore_barrier()
    pltpu.sync_copy(acc.at[pl.ds(sub * RPT, RPT)],
                    out_hbm.at[pl.ds(c * NP + sub * RPT, RPT)])
    plsc.subcore_barrier()


def _sc_agg32(ytab, pk4, zeros32):
  return pl.kernel(
      _agg32_kernel,
      out_type=jax.ShapeDtypeStruct((NCHUNK * NP, CW), jnp.float32),
      mesh=_mesh(),
      compiler_params=pltpu.CompilerParams(use_tc_tiling_on_sc=False),
      scratch_types=[
          pltpu.VMEM((2, KI, 2, B), jnp.int32),
          pltpu.VMEM((GD, B, CW), jnp.float32),
          pltpu.VMEM_SHARED((NP, CW), jnp.float32),
          pltpu.SemaphoreType.DMA,
          pltpu.SemaphoreType.DMA,
          pltpu.SemaphoreType.DMA,
          pltpu.SemaphoreType.DMA,
          pltpu.SemaphoreType.DMA,
      ],
  )(ytab, pk4, zeros32)


def _agg16_kernel(ytab_hbm, pk1_hbm, zeros_hbm, out_hbm, ibuf, rows_v, acc,
                  isem, g0, g1, g2, g3):
  core = lax.axis_index("c")
  sub = lax.axis_index("s")
  tile = core * NS + sub
  pltpu.sync_copy(zeros_hbm.at[pl.ds(sub * RPT, RPT)],
                  acc.at[pl.ds(sub * RPT, RPT)])
  plsc.subcore_barrier()
  _edge_pipeline(pk1_hbm, ytab_hbm, acc, ibuf, rows_v, isem,
                 (g0, g1, g2, g3), tile * NB_HALF, NB_HALF)
  plsc.subcore_barrier()
  pltpu.sync_copy(acc.at[pl.ds(sub * RPT, RPT)],
                  out_hbm.at[pl.ds(core * NP + sub * RPT, RPT)])


def _sc_agg16(ytab, pk1, zeros16):
  return pl.kernel(
      _agg16_kernel,
      out_type=jax.ShapeDtypeStruct((NC * NP, W3P), jnp.float32),
      mesh=_mesh(),
      compiler_params=pltpu.CompilerParams(use_tc_tiling_on_sc=False),
      scratch_types=[
          pltpu.VMEM((2, KI, 2, B), jnp.int32),
          pltpu.VMEM((GD, B, W3P), jnp.float32),
          pltpu.VMEM_SHARED((NP, W3P), jnp.float32),
          pltpu.SemaphoreType.DMA,
          pltpu.SemaphoreType.DMA,
          pltpu.SemaphoreType.DMA,
          pltpu.SemaphoreType.DMA,
          pltpu.SemaphoreType.DMA,
      ],
  )(ytab, pk1, zeros16)


def _chunked(y):
  zpad = jnp.zeros((y.shape[0], CW - D_HID // NCHUNK), jnp.float32)
  parts = []
  for c in range(NCHUNK):
    parts.append(
        jnp.concatenate([y[:, c * 28:(c + 1) * 28], zpad], axis=1)[None])
  return jnp.concatenate(parts, axis=0)


def _norm_from_deg(degp):
  deg = degp[0, :, 0] + degp[1, :, 0]
  return jnp.where(deg > 0.0, lax.rsqrt(jnp.maximum(deg, 1.0)), 0.0)


def _tc1_body(degp_ref, x_ref, w_ref, y0_ref, norm_ref):
  norm = _norm_from_deg(degp_ref[...])
  xs = x_ref[...] * norm[:, None]
  y = jnp.dot(xs, w_ref[...], preferred_element_type=jnp.float32,
              precision=lax.Precision.HIGHEST)
  y0_ref[...] = _chunked(y)
  norm_ref[...] = jnp.broadcast_to(norm[:, None], (RB, 8))


def _tc1(degp, features, w0):
  grid = N // RB
  return pl.pallas_call(
      _tc1_body,
      grid=(grid,),
      in_specs=[
          pl.BlockSpec((2, RB, W3P), lambda i: (0, i, 0)),
          pl.BlockSpec((RB, D_IN), lambda i: (i, 0)),
          pl.BlockSpec((D_IN, D_HID), lambda i: (0, 0)),
      ],
      out_specs=[
          pl.BlockSpec((NCHUNK, RB, CW), lambda i: (0, i, 0)),
          pl.BlockSpec((RB, 8), lambda i: (i, 0)),
      ],
      out_shape=[
          jax.ShapeDtypeStruct((NCHUNK, N, CW), jnp.float32),
          jax.ShapeDtypeStruct((N, 8), jnp.float32),
      ],
  )(degp, features, w0)


def _mid_body(residual, use_act, agg_ref, norm_ref, w_ref, *rest):
  if residual:
    xin_ref = rest[0]
    rest = rest[1:]
  x_ref, y_ref = rest
  a = agg_ref[...]
  a112 = jnp.concatenate([a[c, :, :28] for c in range(NCHUNK)], axis=1)
  n = norm_ref[:, 0]
  h = a112 * n[:, None]
  if use_act:
    h = jnp.tanh(h)
  if residual:
    h = h + xin_ref[...]
  y = jnp.dot(h * n[:, None], w_ref[...], preferred_element_type=jnp.float32,
              precision=lax.Precision.HIGHEST)
  x_ref[...] = h
  y_ref[...] = _chunked(y)


def _tc_mid(agg, normw, w, xin):
  grid = N // RB
  residual = xin is not None
  body = functools.partial(_mid_body, residual, not residual)
  in_specs = [
      pl.BlockSpec((NCHUNK, RB, CW), lambda i: (0, i, 0)),
      pl.BlockSpec((RB, 8), lambda i: (i, 0)),
      pl.BlockSpec((D_HID, D_HID), lambda i: (0, 0)),
  ]
  args = [agg, normw, w]
  if residual:
    in_specs.append(pl.BlockSpec((RB, D_HID), lambda i: (i, 0)))
    args.append(xin)
  return pl.pallas_call(
      body,
      grid=(grid,),
      in_specs=in_specs,
      out_specs=[
          pl.BlockSpec((RB, D_HID), lambda i: (i, 0)),
          pl.BlockSpec((NCHUNK, RB, CW), lambda i: (0, i, 0)),
      ],
      out_shape=[
          jax.ShapeDtypeStruct((N, D_HID), jnp.float32),
          jax.ShapeDtypeStruct((NCHUNK, N, CW), jnp.float32),
      ],
  )(*args)


def _tc4_body(agg_ref, norm_ref, xin_ref, w_ref, y_ref):
  a = agg_ref[...]
  a112 = jnp.concatenate([a[c, :, :28] for c in range(NCHUNK)], axis=1)
  n = norm_ref[:, 0]
  h = a112 * n[:, None] + xin_ref[...]
  y_ref[...] = jnp.dot(h * n[:, None], w_ref[...],
                       preferred_element_type=jnp.float32,
                       precision=lax.Precision.HIGHEST)


def _tc4(agg, normw, xin, w3p):
  grid = N // RB
  return pl.pallas_call(
      _tc4_body,
      grid=(grid,),
      in_specs=[
          pl.BlockSpec((NCHUNK, RB, CW), lambda i: (0, i, 0)),
          pl.BlockSpec((RB, 8), lambda i: (i, 0)),
          pl.BlockSpec((RB, D_HID), lambda i: (i, 0)),
          pl.BlockSpec((D_HID, W3P), lambda i: (0, 0)),
      ],
      out_specs=pl.BlockSpec((RB, W3P), lambda i: (i, 0)),
      out_shape=jax.ShapeDtypeStruct((N, W3P), jnp.float32),
  )(agg, normw, xin, w3p)


def _tc5_body(aggp_ref, norm_ref, out_ref):
  a = aggp_ref[0] + aggp_ref[1]
  out_ref[...] = a * norm_ref[:, :1]


def _tc5(aggp, normw):
  grid = N // RB
  return pl.pallas_call(
      _tc5_body,
      grid=(grid,),
      in_specs=[
          pl.BlockSpec((2, RB, W3P), lambda i: (0, i, 0)),
          pl.BlockSpec((RB, 8), lambda i: (i, 0)),
      ],
      out_specs=pl.BlockSpec((RB, W3P), lambda i: (i, 0)),
      out_shape=jax.ShapeDtypeStruct((N, W3P), jnp.float32),
  )(aggp, normw)


@jax.jit
def kernel(features, edge_index, W0, W1, W2, W3):
  src = edge_index[0].astype(jnp.int32)
  dst = edge_index[1].astype(jnp.int32)
  pad_e = E_PAD - E
  srcp = jnp.concatenate([src, jnp.zeros((pad_e,), jnp.int32)])
  dstp = jnp.concatenate([dst, jnp.full((pad_e,), N, jnp.int32)])

  src4 = srcp[None, :] + (jnp.arange(NCHUNK, dtype=jnp.int32) * N)[:, None]
  s4 = src4.reshape(NCHUNK, NS, NB_FULL, B)
  d4 = jnp.broadcast_to(dstp.reshape(1, NS, NB_FULL, B), s4.shape)
  pk4 = jnp.stack([s4, d4], axis=3).reshape(NCHUNK * NS * NB_FULL, 2, B)

  s1 = srcp.reshape(NC * NS, NB_HALF, B)
  d1 = dstp.reshape(NC * NS, NB_HALF, B)
  pk1 = jnp.stack([s1, d1], axis=2).reshape(NC * NS * NB_HALF, 2, B)

  zeros32 = jnp.zeros((NP, CW), jnp.float32)
  zeros16 = jnp.zeros((NP, W3P), jnp.float32)
  ones16 = jnp.ones((B, W3P), jnp.float32)
  w3p = jnp.pad(W3, ((0, 0), (0, W3P - D_OUT)))

  degp = _sc_deg(pk1, ones16, zeros16).reshape(NC, NP, W3P)
  y0, normw = _tc1(degp, features, W0)
  agg0 = _sc_agg32(y0.reshape(NCHUNK * N, CW), pk4, zeros32)
  x1, y1 = _tc_mid(agg0.reshape(NCHUNK, NP, CW), normw, W1, None)
  agg1 = _sc_agg32(y1.reshape(NCHUNK * N, CW), pk4, zeros32)
  x2, y2 = _tc_mid(agg1.reshape(NCHUNK, NP, CW), normw, W2, x1)
  agg2 = _sc_agg32(y2.reshape(NCHUNK * N, CW), pk4, zeros32)
  y3 = _tc4(agg2.reshape(NCHUNK, NP, CW), normw, x2, w3p)
  aggp3 = _sc_agg16(y3, pk1, zeros16)
  out = _tc5(aggp3.reshape(NC, NP, W3P), normw)
  return out[:, :D_OUT]


# speedup vs baseline: 1.3932x; 1.0800x over previous
import functools

import jax
import jax.numpy as jnp
from jax import lax
from jax.experimental import pallas as pl
from jax.experimental.pallas import tpu as pltpu
from jax.experimental.pallas import tpu_sc as plsc

N = 50000
E = 800000
D_IN = 1433
D_HID = 112
D_OUT = 7

NC = 2
NS = 16
B = 128
NP = 50048
RPT = NP // NS
E_PAD = 819200
NB_FULL = E_PAD // (NS * B)
NB_HALF = E_PAD // (NC * NS * B)
CW = 32
NCHUNK = 4
W3P = 16
RB = 400

@functools.lru_cache(maxsize=1)
def _mesh():
  return plsc.VectorSubcoreMesh(
      core_axis_name="c", subcore_axis_name="s", num_cores=NC, num_subcores=NS)


def _deg_kernel(pk1_hbm, ones_hbm, zeros_hbm, out_hbm, idx_v, ones_v, acc):
  core = lax.axis_index("c")
  sub = lax.axis_index("s")
  tile = core * NS + sub
  pltpu.sync_copy(zeros_hbm.at[pl.ds(sub * RPT, RPT)],
                  acc.at[pl.ds(sub * RPT, RPT)])
  pltpu.sync_copy(ones_hbm, ones_v)
  plsc.subcore_barrier()

  def body(t, carry):
    row = tile * NB_HALF + t
    pltpu.sync_copy(pk1_hbm.at[row], idx_v)
    pltpu.sync_copy(ones_v, acc.at[idx_v.at[1]], add=True)
    return carry

  lax.fori_loop(0, NB_HALF, body, 0)
  plsc.subcore_barrier()
  pltpu.sync_copy(acc.at[pl.ds(sub * RPT, RPT)],
                  out_hbm.at[pl.ds(core * NP + sub * RPT, RPT)])


def _sc_deg(pk1, ones16, zeros16):
  return pl.kernel(
      _deg_kernel,
      out_type=jax.ShapeDtypeStruct((NC * NP, W3P), jnp.float32),
      mesh=_mesh(),
      compiler_params=pltpu.CompilerParams(use_tc_tiling_on_sc=False),
      scratch_types=[
          pltpu.VMEM((2, B), jnp.int32),
          pltpu.VMEM((B, W3P), jnp.float32),
          pltpu.VMEM_SHARED((NP, W3P), jnp.float32),
      ],
  )(pk1, ones16, zeros16)


KI = 8
GD = 4


def _edge_pipeline(pk_hbm, ytab_hbm, acc, ibuf, rows_v, isem, gsems, rowbase,
                   nb):
  ng = nb // KI

  def idx_start(g, buf):
    pltpu.async_copy(pk_hbm.at[pl.ds(rowbase + g * KI, KI)], ibuf.at[buf],
                     isem)

  def idx_wait(buf):
    pltpu.make_async_copy(pk_hbm.at[pl.ds(rowbase, KI)], ibuf.at[buf],
                          isem).wait()

  def gather_start(buf, k, slot):
    pltpu.async_copy(ytab_hbm.at[ibuf.at[buf, k, 0]], rows_v.at[slot],
                     gsems[slot])

  def gather_wait(buf, k, slot):
    pltpu.make_async_copy(ytab_hbm.at[ibuf.at[buf, k, 0]], rows_v.at[slot],
                          gsems[slot]).wait()

  idx_start(0, 0)
  idx_wait(0)
  gather_start(0, 0, 0)
  gather_start(0, 1, 1)
  gather_start(0, 2, 2)

  def outer(g, carry):
    gm = lax.rem(g, 2)
    gn = lax.rem(g + 1, 2)

    @pl.when(g + 1 < ng)
    def _():
      idx_start(g + 1, gn)

    for k in range(KI):
      t = g * KI + k

      @pl.when(t + 3 < nb)
      def _():
        if k < KI - 3:
          gather_start(gm, k + 3, (k + 3) % GD)
        else:
          if k == KI - 3:
            idx_wait(gn)
          gather_start(gn, k + 3 - KI, (k + 3) % GD)

      gather_wait(gm, k, k % GD)
      pltpu.sync_copy(rows_v.at[k % GD], acc.at[ibuf.at[gm, k, 1]], add=True)
    return carry

  lax.fori_loop(0, ng, outer, 0)


def _agg32_kernel(ytab_hbm, pk4_hbm, zeros_hbm, out_hbm, ibuf, rows_v, acc,
                  isem, g0, g1, g2, g3):
  core = lax.axis_index("c")
  sub = lax.axis_index("s")
  for j in range(NCHUNK // NC):
    c = core * (NCHUNK // NC) + j
    pltpu.sync_copy(zeros_hbm.at[pl.ds(sub * RPT, RPT)],
                    acc.at[pl.ds(sub * RPT, RPT)])
    plsc.subcore_barrier()
    rowbase = sub * NB_FULL
    _edge_pipeline(pk4_hbm, ytab_hbm.at[pl.ds(c * N, N)], acc, ibuf, rows_v,
                   isem, (g0, g1, g2, g3), rowbase, NB_FULL)
    plsc.subcore_barrier()
    pltpu.sync_copy(acc.at[pl.ds(sub * RPT, RPT)],
                    out_hbm.at[pl.ds(c * NP + sub * RPT, RPT)])
    plsc.subcore_barrier()


def _sc_agg32(ytab, pk4, zeros32):
  return pl.kernel(
      _agg32_kernel,
      out_type=jax.ShapeDtypeStruct((NCHUNK * NP, CW), jnp.float32),
      mesh=_mesh(),
      compiler_params=pltpu.CompilerParams(use_tc_tiling_on_sc=False),
      scratch_types=[
          pltpu.VMEM((2, KI, 2, B), jnp.int32),
          pltpu.VMEM((GD, B, CW), jnp.float32),
          pltpu.VMEM_SHARED((NP, CW), jnp.float32),
          pltpu.SemaphoreType.DMA,
          pltpu.SemaphoreType.DMA,
          pltpu.SemaphoreType.DMA,
          pltpu.SemaphoreType.DMA,
          pltpu.SemaphoreType.DMA,
      ],
  )(ytab, pk4, zeros32)


def _agg16_kernel(ytab_hbm, pk1_hbm, zeros_hbm, out_hbm, ibuf, rows_v, acc,
                  isem, g0, g1, g2, g3):
  core = lax.axis_index("c")
  sub = lax.axis_index("s")
  tile = core * NS + sub
  pltpu.sync_copy(zeros_hbm.at[pl.ds(sub * RPT, RPT)],
                  acc.at[pl.ds(sub * RPT, RPT)])
  plsc.subcore_barrier()
  _edge_pipeline(pk1_hbm, ytab_hbm, acc, ibuf, rows_v, isem,
                 (g0, g1, g2, g3), tile * NB_HALF, NB_HALF)
  plsc.subcore_barrier()
  pltpu.sync_copy(acc.at[pl.ds(sub * RPT, RPT)],
                  out_hbm.at[pl.ds(core * NP + sub * RPT, RPT)])


def _sc_agg16(ytab, pk1, zeros16):
  return pl.kernel(
      _agg16_kernel,
      out_type=jax.ShapeDtypeStruct((NC * NP, W3P), jnp.float32),
      mesh=_mesh(),
      compiler_params=pltpu.CompilerParams(use_tc_tiling_on_sc=False),
      scratch_types=[
          pltpu.VMEM((2, KI, 2, B), jnp.int32),
          pltpu.VMEM((GD, B, W3P), jnp.float32),
          pltpu.VMEM_SHARED((NP, W3P), jnp.float32),
          pltpu.SemaphoreType.DMA,
          pltpu.SemaphoreType.DMA,
          pltpu.SemaphoreType.DMA,
          pltpu.SemaphoreType.DMA,
          pltpu.SemaphoreType.DMA,
      ],
  )(ytab, pk1, zeros16)


def _chunked(y):
  zpad = jnp.zeros((y.shape[0], CW - D_HID // NCHUNK), jnp.float32)
  parts = []
  for c in range(NCHUNK):
    parts.append(
        jnp.concatenate([y[:, c * 28:(c + 1) * 28], zpad], axis=1)[None])
  return jnp.concatenate(parts, axis=0)


def _norm_from_deg(degp):
  deg = degp[0, :, 0] + degp[1, :, 0]
  return jnp.where(deg > 0.0, lax.rsqrt(jnp.maximum(deg, 1.0)), 0.0)


def _tc1_body(degp_ref, x_ref, w_ref, y0_ref, norm_ref):
  norm = _norm_from_deg(degp_ref[...])
  xs = x_ref[...] * norm[:, None]
  y = jnp.dot(xs, w_ref[...], preferred_element_type=jnp.float32)
  y0_ref[...] = _chunked(y)
  norm_ref[...] = jnp.broadcast_to(norm[:, None], (RB, 8))


def _tc1(degp, features, w0):
  grid = N // RB
  return pl.pallas_call(
      _tc1_body,
      grid=(grid,),
      in_specs=[
          pl.BlockSpec((2, RB, W3P), lambda i: (0, i, 0)),
          pl.BlockSpec((RB, D_IN), lambda i: (i, 0)),
          pl.BlockSpec((D_IN, D_HID), lambda i: (0, 0)),
      ],
      out_specs=[
          pl.BlockSpec((NCHUNK, RB, CW), lambda i: (0, i, 0)),
          pl.BlockSpec((RB, 8), lambda i: (i, 0)),
      ],
      out_shape=[
          jax.ShapeDtypeStruct((NCHUNK, N, CW), jnp.float32),
          jax.ShapeDtypeStruct((N, 8), jnp.float32),
      ],
  )(degp, features, w0)


def _mid_body(residual, use_act, agg_ref, norm_ref, w_ref, *rest):
  if residual:
    xin_ref = rest[0]
    rest = rest[1:]
  x_ref, y_ref = rest
  a = agg_ref[...]
  a112 = jnp.concatenate([a[c, :, :28] for c in range(NCHUNK)], axis=1)
  n = norm_ref[:, 0]
  h = a112 * n[:, None]
  if use_act:
    h = jnp.tanh(h)
  if residual:
    h = h + xin_ref[...]
  y = jnp.dot(h * n[:, None], w_ref[...], preferred_element_type=jnp.float32)
  x_ref[...] = h
  y_ref[...] = _chunked(y)


def _tc_mid(agg, normw, w, xin):
  grid = N // RB
  residual = xin is not None
  body = functools.partial(_mid_body, residual, not residual)
  in_specs = [
      pl.BlockSpec((NCHUNK, RB, CW), lambda i: (0, i, 0)),
      pl.BlockSpec((RB, 8), lambda i: (i, 0)),
      pl.BlockSpec((D_HID, D_HID), lambda i: (0, 0)),
  ]
  args = [agg, normw, w]
  if residual:
    in_specs.append(pl.BlockSpec((RB, D_HID), lambda i: (i, 0)))
    args.append(xin)
  return pl.pallas_call(
      body,
      grid=(grid,),
      in_specs=in_specs,
      out_specs=[
          pl.BlockSpec((RB, D_HID), lambda i: (i, 0)),
          pl.BlockSpec((NCHUNK, RB, CW), lambda i: (0, i, 0)),
      ],
      out_shape=[
          jax.ShapeDtypeStruct((N, D_HID), jnp.float32),
          jax.ShapeDtypeStruct((NCHUNK, N, CW), jnp.float32),
      ],
  )(*args)


def _tc4_body(agg_ref, norm_ref, xin_ref, w_ref, y_ref):
  a = agg_ref[...]
  a112 = jnp.concatenate([a[c, :, :28] for c in range(NCHUNK)], axis=1)
  n = norm_ref[:, 0]
  h = a112 * n[:, None] + xin_ref[...]
  y_ref[...] = jnp.dot(h * n[:, None], w_ref[...],
                       preferred_element_type=jnp.float32)


def _tc4(agg, normw, xin, w3p):
  grid = N // RB
  return pl.pallas_call(
      _tc4_body,
      grid=(grid,),
      in_specs=[
          pl.BlockSpec((NCHUNK, RB, CW), lambda i: (0, i, 0)),
          pl.BlockSpec((RB, 8), lambda i: (i, 0)),
          pl.BlockSpec((RB, D_HID), lambda i: (i, 0)),
          pl.BlockSpec((D_HID, W3P), lambda i: (0, 0)),
      ],
      out_specs=pl.BlockSpec((RB, W3P), lambda i: (i, 0)),
      out_shape=jax.ShapeDtypeStruct((N, W3P), jnp.float32),
  )(agg, normw, xin, w3p)


def _tc5_body(aggp_ref, norm_ref, out_ref):
  a = aggp_ref[0] + aggp_ref[1]
  out_ref[...] = a * norm_ref[:, :1]


def _tc5(aggp, normw):
  grid = N // RB
  return pl.pallas_call(
      _tc5_body,
      grid=(grid,),
      in_specs=[
          pl.BlockSpec((2, RB, W3P), lambda i: (0, i, 0)),
          pl.BlockSpec((RB, 8), lambda i: (i, 0)),
      ],
      out_specs=pl.BlockSpec((RB, W3P), lambda i: (i, 0)),
      out_shape=jax.ShapeDtypeStruct((N, W3P), jnp.float32),
  )(aggp, normw)


@jax.jit
def kernel(features, edge_index, W0, W1, W2, W3):
  src = edge_index[0].astype(jnp.int32)
  dst = edge_index[1].astype(jnp.int32)
  pad_e = E_PAD - E
  srcp = jnp.concatenate([src, jnp.zeros((pad_e,), jnp.int32)])
  dstp = jnp.concatenate([dst, jnp.full((pad_e,), N, jnp.int32)])

  s1 = srcp.reshape(NC * NS, NB_HALF, B)
  d1 = dstp.reshape(NC * NS, NB_HALF, B)
  pk1 = jnp.stack([s1, d1], axis=2).reshape(NC * NS * NB_HALF, 2, B)

  zeros32 = jnp.zeros((NP, CW), jnp.float32)
  zeros16 = jnp.zeros((NP, W3P), jnp.float32)
  ones16 = jnp.ones((B, W3P), jnp.float32)
  w3p = jnp.pad(W3, ((0, 0), (0, W3P - D_OUT)))

  degp = _sc_deg(pk1, ones16, zeros16).reshape(NC, NP, W3P)
  y0, normw = _tc1(degp, features, W0)
  agg0 = _sc_agg32(y0.reshape(NCHUNK * N, CW), pk1, zeros32)
  x1, y1 = _tc_mid(agg0.reshape(NCHUNK, NP, CW), normw, W1, None)
  agg1 = _sc_agg32(y1.reshape(NCHUNK * N, CW), pk1, zeros32)
  x2, y2 = _tc_mid(agg1.reshape(NCHUNK, NP, CW), normw, W2, x1)
  agg2 = _sc_agg32(y2.reshape(NCHUNK * N, CW), pk1, zeros32)
  y3 = _tc4(agg2.reshape(NCHUNK, NP, CW), normw, x2, w3p)
  aggp3 = _sc_agg16(y3, pk1, zeros16)
  out = _tc5(aggp3.reshape(NC, NP, W3P), normw)
  return out[:, :D_OUT]
